# baseline (speedup 1.0000x reference)
import jax
import jax.numpy as jnp
import numpy as np
from jax import lax
from jax.experimental import pallas as pl
from jax.experimental.pallas import tpu as pltpu
from jax.experimental.pallas import tpu_sc as plsc

NOTES_POOL_SIZE = 1000000
EMBED_DIM = 64
BATCH = 4096
SEQ_LEN = 200

NC = 2
NS = 16
NW = NC * NS
SLAB = 128
NFULL = NOTES_POOL_SIZE // SLAB
TAIL = NOTES_POOL_SIZE - NFULL * SLAB
T2_ROWS = (NFULL + 1) * SLAB
TAIL_WORKER = NFULL % NW


def _positional_encoding(max_pos, embed_dim):
    pos = np.arange(max_pos)[:, np.newaxis]
    i = np.arange(embed_dim)[np.newaxis, :]
    angle_rates = 1.0 / np.power(10000, 2 * (i // 2) / np.float32(embed_dim))
    angle_rads = pos * angle_rates
    angle_rads[:, 0::2] = np.sin(angle_rads[:, 0::2])
    angle_rads[:, 1::2] = np.cos(angle_rads[:, 1::2])
    return angle_rads.astype(np.float32)


def _splat(v):
    return jnp.full((16,), v, jnp.int32)


def _transpose_body(tt_hbm, tail_hbm, t2_hbm, slab0, slab1, tbuf0, tbuf1,
                    tail_v, isem0, isem1, osem0, osem1, tsem):
    wid = lax.axis_index("s") * NC + lax.axis_index("c")
    slab = [slab0, slab1]
    tbuf = [tbuf0, tbuf1]
    isem = [isem0, isem1]
    osem = [osem0, osem1]
    iota = lax.iota(jnp.int32, 16)
    iotas = [iota + 16 * k for k in range(EMBED_DIM // 16)]

    nmine = (NFULL - wid + NW - 1) // NW

    def col_ds(j):
        return pl.ds(pl.multiple_of(j * SLAB, SLAB), SLAB)

    def start_in(i, p):
        j = wid + NW * i
        pltpu.async_copy(tt_hbm.at[:, col_ds(j)], slab[p], isem[p])

    def wait_in(p):
        pltpu.make_async_copy(tt_hbm.at[:, col_ds(0)], slab[p],
                              isem[p]).wait()

    def start_out(i, p):
        j = wid + NW * i
        pltpu.async_copy(tbuf[p], t2_hbm.at[col_ds(j)], osem[p])

    def wait_out(p):
        pltpu.make_async_copy(tbuf[p], t2_hbm.at[col_ds(0)], osem[p]).wait()

    def transpose_slab(p):
        def row(r, carry):
            for rr in range(2):
                c = 2 * r + rr
                sc = _splat(c)
                for k in range(EMBED_DIM // 16):
                    v = plsc.load_gather(slab[p], [iotas[k], sc])
                    tbuf[p][c, pl.ds(16 * k, 16)] = v
            return carry
        lax.fori_loop(0, SLAB // 2, row, 0)

    @pl.when(nmine > 0)
    def _():
        start_in(0, 0)

    def step(i, carry):
        for p in range(2):
            @pl.when(lax.rem(i, 2) == p)
            def _(p=p):
                wait_in(p)

                @pl.when(i + 1 < nmine)
                def _():
                    start_in(i + 1, 1 - p)

                @pl.when(i >= 2)
                def _():
                    wait_out(p)

                transpose_slab(p)
                start_out(i, p)
        return carry

    lax.fori_loop(0, nmine, step, 0)

    wait_out(0)
    wait_out(1)

    @pl.when(wid == TAIL_WORKER)
    def _():
        pltpu.async_copy(tail_hbm, tail_v, tsem)
        pltpu.make_async_copy(tail_hbm, tail_v, tsem).wait()

        def row(r, carry):
            for k in range(EMBED_DIM // 16):
                s16 = pl.ds(16 * k, 16)
                tbuf0[r, s16] = tail_v[r, s16]
            return carry
        lax.fori_loop(0, TAIL, row, 0)
        pltpu.async_copy(tbuf0.at[pl.ds(0, TAIL)],
                         t2_hbm.at[pl.ds(NFULL * SLAB, TAIL)], tsem)
        pltpu.make_async_copy(tbuf0.at[pl.ds(0, TAIL)],
                              t2_hbm.at[pl.ds(NFULL * SLAB, TAIL)],
                              tsem).wait()


def _gather_body(xt_hbm, t2_hbm, pos_hbm, out_hbm,
                 idx_v, pos_v, g0, g1, g2, ob0, ob1,
                 gsem0, gsem1, gsem2, osem0, osem1):
    wid = lax.axis_index("s") * NC + lax.axis_index("c")
    g = [g0, g1, g2]
    ob = [ob0, ob1]
    gsem = [gsem0, gsem1, gsem2]
    osem = [osem0, osem1]
    iota = lax.iota(jnp.int32, 16)
    iotas = [iota + 16 * m for m in range(SLAB // 16)]

    bds = pl.ds(pl.multiple_of(wid * SLAB, SLAB), SLAB)
    pltpu.sync_copy(xt_hbm.at[:, bds], idx_v)
    pltpu.sync_copy(pos_hbm, pos_v)

    def fire_gather(s, q):
        pltpu.async_copy(t2_hbm.at[idx_v.at[s]], g[q], gsem[q])

    def wait_gather(s, q):
        pltpu.make_async_copy(t2_hbm.at[idx_v.at[s]], g[q], gsem[q]).wait()

    def wait_store(s, p):
        for dt in range(EMBED_DIM // 8):
            pltpu.make_async_copy(ob[p].at[pl.ds(8 * dt, 8)],
                                  out_hbm.at[s, dt, wid], osem[p]).wait()

    fire_gather(0, 0)

    def step(s, carry):
        q0 = lax.rem(s, 3)
        p0 = lax.rem(s, 2)
        for q in range(3):
            @pl.when(q0 == q)
            def _(q=q):
                wait_gather(s, q)

                @pl.when(s + 1 < SEQ_LEN)
                def _():
                    fire_gather(s + 1, (q + 1) % 3)

                for p in range(2):
                    @pl.when(p0 == p)
                    def _(p=p, q=q):
                        @pl.when(s >= 2)
                        def _():
                            wait_store(s - 2, p)

                        ss = _splat(s)

                        def drow(d, carry):
                            sd = _splat(d)
                            pv = plsc.load_gather(pos_v, [ss, sd])
                            for m in range(SLAB // 16):
                                v = plsc.load_gather(g[q], [iotas[m], sd])
                                ob[p][d, pl.ds(16 * m, 16)] = v + pv
                            return carry
                        lax.fori_loop(0, EMBED_DIM, drow, 0)

                        for dt in range(EMBED_DIM // 8):
                            pltpu.async_copy(ob[p].at[pl.ds(8 * dt, 8)],
                                             out_hbm.at[s, dt, wid], osem[p])
        return carry

    lax.fori_loop(0, SEQ_LEN, step, 0)
    wait_store(SEQ_LEN - 2, 0)
    wait_store(SEQ_LEN - 1, 1)


@jax.jit
def _run(xt, tt, tail, pos):
    mesh = plsc.VectorSubcoreMesh(core_axis_name="c", subcore_axis_name="s")
    t2 = pl.kernel(
        _transpose_body,
        out_type=jax.ShapeDtypeStruct((T2_ROWS, SLAB), jnp.float32),
        mesh=mesh,
        scratch_types=[
            pltpu.VMEM((EMBED_DIM, SLAB), jnp.float32),
            pltpu.VMEM((EMBED_DIM, SLAB), jnp.float32),
            pltpu.VMEM((SLAB, SLAB), jnp.float32),
            pltpu.VMEM((SLAB, SLAB), jnp.float32),
            pltpu.VMEM((TAIL, EMBED_DIM), jnp.float32),
            pltpu.SemaphoreType.DMA,
            pltpu.SemaphoreType.DMA,
            pltpu.SemaphoreType.DMA,
            pltpu.SemaphoreType.DMA,
            pltpu.SemaphoreType.DMA,
        ],
        compiler_params=pltpu.CompilerParams(needs_layout_passes=False),
    )(tt, tail)

    y5 = pl.kernel(
        _gather_body,
        out_type=jax.ShapeDtypeStruct(
            (SEQ_LEN, EMBED_DIM // 8, NW, 8, SLAB), jnp.float32),
        mesh=mesh,
        scratch_types=[
            pltpu.VMEM((SEQ_LEN, SLAB), jnp.int32),
            pltpu.VMEM((SEQ_LEN, EMBED_DIM), jnp.float32),
            pltpu.VMEM((SLAB, SLAB), jnp.float32),
            pltpu.VMEM((SLAB, SLAB), jnp.float32),
            pltpu.VMEM((SLAB, SLAB), jnp.float32),
            pltpu.VMEM((EMBED_DIM, SLAB), jnp.float32),
            pltpu.VMEM((EMBED_DIM, SLAB), jnp.float32),
            pltpu.SemaphoreType.DMA,
            pltpu.SemaphoreType.DMA,
            pltpu.SemaphoreType.DMA,
            pltpu.SemaphoreType.DMA,
            pltpu.SemaphoreType.DMA,
        ],
        compiler_params=pltpu.CompilerParams(needs_layout_passes=False),
    )(xt, t2, pos)
    return y5


def kernel(x_in, table):
    xt = x_in.astype(jnp.int32).T
    tt = table.T
    tail = table[NFULL * SLAB:]
    pos = jnp.asarray(_positional_encoding(SEQ_LEN, EMBED_DIM))
    y5 = _run(xt, tt, tail, pos)
    return y5.transpose(2, 4, 0, 1, 3).reshape(BATCH, SEQ_LEN, EMBED_DIM)


# speedup vs baseline: 2.0061x; 2.0061x over previous
import jax
import jax.numpy as jnp
import numpy as np
from jax import lax
from jax.experimental import pallas as pl
from jax.experimental.pallas import tpu as pltpu
from jax.experimental.pallas import tpu_sc as plsc

NOTES_POOL_SIZE = 1000000
EMBED_DIM = 64
BATCH = 4096
SEQ_LEN = 200

NC = 2
NS = 16
NW = NC * NS
SLAB = 128
NFULL = NOTES_POOL_SIZE // SLAB
TAIL = NOTES_POOL_SIZE - NFULL * SLAB
T2_ROWS = (NFULL + 1) * SLAB
TAIL_WORKER = NFULL % NW


def _positional_encoding(max_pos, embed_dim):
    pos = np.arange(max_pos)[:, np.newaxis]
    i = np.arange(embed_dim)[np.newaxis, :]
    angle_rates = 1.0 / np.power(10000, 2 * (i // 2) / np.float32(embed_dim))
    angle_rads = pos * angle_rates
    angle_rads[:, 0::2] = np.sin(angle_rads[:, 0::2])
    angle_rads[:, 1::2] = np.cos(angle_rads[:, 1::2])
    return angle_rads.astype(np.float32)


def _splat(v):
    return jnp.full((16,), v, jnp.int32)


def _transpose_body(tt_hbm, tail_hbm, t2_hbm, slab0, slab1, tbuf0, tbuf1,
                    tail_v, isem0, isem1, osem0, osem1, tsem):
    wid = lax.axis_index("s") * NC + lax.axis_index("c")
    slab = [slab0, slab1]
    tbuf = [tbuf0, tbuf1]
    isem = [isem0, isem1]
    osem = [osem0, osem1]
    iota = lax.iota(jnp.int32, 16)
    iotas = [iota + 16 * k for k in range(EMBED_DIM // 16)]

    nmine = (NFULL - wid + NW - 1) // NW

    def col_ds(j):
        return pl.ds(pl.multiple_of(j * SLAB, SLAB), SLAB)

    def start_in(i, p):
        j = wid + NW * i
        pltpu.async_copy(tt_hbm.at[:, col_ds(j)], slab[p], isem[p])

    def wait_in(p):
        pltpu.make_async_copy(tt_hbm.at[:, col_ds(0)], slab[p],
                              isem[p]).wait()

    def start_out(i, p):
        j = wid + NW * i
        pltpu.async_copy(tbuf[p], t2_hbm.at[col_ds(j)], osem[p])

    def wait_out(p):
        pltpu.make_async_copy(tbuf[p], t2_hbm.at[col_ds(0)], osem[p]).wait()

    diags = [lax.rem(iota + i, 16) for i in range(16)]

    def transpose_slab(p):
        def cblock(m, carry):
            cvec = iota + 16 * m
            for k in range(EMBED_DIM // 16):
                for i in range(16):
                    dvec = diags[i] + (16 * k)
                    v = plsc.load_gather(slab[p], [dvec, cvec])
                    plsc.store_scatter(tbuf[p], [cvec, dvec], v)
            return carry
        lax.fori_loop(0, SLAB // 16, cblock, 0)

    @pl.when(nmine > 0)
    def _():
        start_in(0, 0)

    def step(i, carry):
        for p in range(2):
            @pl.when(lax.rem(i, 2) == p)
            def _(p=p):
                wait_in(p)

                @pl.when(i + 1 < nmine)
                def _():
                    start_in(i + 1, 1 - p)

                @pl.when(i >= 2)
                def _():
                    wait_out(p)

                transpose_slab(p)
                start_out(i, p)
        return carry

    lax.fori_loop(0, nmine, step, 0)

    wait_out(0)
    wait_out(1)

    @pl.when(wid == TAIL_WORKER)
    def _():
        pltpu.async_copy(tail_hbm, tail_v, tsem)
        pltpu.make_async_copy(tail_hbm, tail_v, tsem).wait()

        def row(r, carry):
            for k in range(EMBED_DIM // 16):
                s16 = pl.ds(16 * k, 16)
                tbuf0[r, s16] = tail_v[r, s16]
            return carry
        lax.fori_loop(0, TAIL, row, 0)
        pltpu.async_copy(tbuf0.at[pl.ds(0, TAIL)],
                         t2_hbm.at[pl.ds(NFULL * SLAB, TAIL)], tsem)
        pltpu.make_async_copy(tbuf0.at[pl.ds(0, TAIL)],
                              t2_hbm.at[pl.ds(NFULL * SLAB, TAIL)],
                              tsem).wait()


def _gather_body(xt_hbm, t2_hbm, pos_hbm, out_hbm,
                 idx_v, pos_v, g0, g1, g2, ob0, ob1,
                 gsem0, gsem1, gsem2, osem0, osem1):
    wid = lax.axis_index("s") * NC + lax.axis_index("c")
    g = [g0, g1, g2]
    ob = [ob0, ob1]
    gsem = [gsem0, gsem1, gsem2]
    osem = [osem0, osem1]
    iota = lax.iota(jnp.int32, 16)
    diags = [lax.rem(iota + i, 16) for i in range(16)]

    bds = pl.ds(pl.multiple_of(wid * SLAB, SLAB), SLAB)
    pltpu.sync_copy(xt_hbm.at[:, bds], idx_v)
    pltpu.sync_copy(pos_hbm, pos_v)

    def fire_gather(s, q):
        pltpu.async_copy(t2_hbm.at[idx_v.at[s]], g[q], gsem[q])

    def wait_gather(s, q):
        pltpu.make_async_copy(t2_hbm.at[idx_v.at[s]], g[q], gsem[q]).wait()

    def wait_store(s, p):
        for dt in range(EMBED_DIM // 8):
            pltpu.make_async_copy(ob[p].at[pl.ds(8 * dt, 8)],
                                  out_hbm.at[s, dt, wid], osem[p]).wait()

    fire_gather(0, 0)

    def step(s, carry):
        q0 = lax.rem(s, 3)
        p0 = lax.rem(s, 2)
        for q in range(3):
            @pl.when(q0 == q)
            def _(q=q):
                wait_gather(s, q)

                @pl.when(s + 1 < SEQ_LEN)
                def _():
                    fire_gather(s + 1, (q + 1) % 3)

                for p in range(2):
                    @pl.when(p0 == p)
                    def _(p=p, q=q):
                        @pl.when(s >= 2)
                        def _():
                            wait_store(s - 2, p)

                        pregs = [pos_v[s, pl.ds(16 * k, 16)]
                                 for k in range(EMBED_DIM // 16)]

                        def brow(bl, carry):
                            for k in range(EMBED_DIM // 16):
                                plsc.addupdate(
                                    g[q].at[bl, pl.ds(16 * k, 16)],
                                    pregs[k])
                            return carry
                        lax.fori_loop(0, SLAB, brow, 0)

                        def bblock(m, carry):
                            blvec = iota + 16 * m
                            for k in range(EMBED_DIM // 16):
                                for i in range(16):
                                    dvec = diags[i] + (16 * k)
                                    v = plsc.load_gather(g[q], [blvec, dvec])
                                    plsc.store_scatter(ob[p], [dvec, blvec],
                                                       v)
                            return carry
                        lax.fori_loop(0, SLAB // 16, bblock, 0)

                        for dt in range(EMBED_DIM // 8):
                            pltpu.async_copy(ob[p].at[pl.ds(8 * dt, 8)],
                                             out_hbm.at[s, dt, wid], osem[p])
        return carry

    lax.fori_loop(0, SEQ_LEN, step, 0)
    wait_store(SEQ_LEN - 2, 0)
    wait_store(SEQ_LEN - 1, 1)


@jax.jit
def _run(xt, tt, tail, pos):
    mesh = plsc.VectorSubcoreMesh(core_axis_name="c", subcore_axis_name="s")
    t2 = pl.kernel(
        _transpose_body,
        out_type=jax.ShapeDtypeStruct((T2_ROWS, SLAB), jnp.float32),
        mesh=mesh,
        scratch_types=[
            pltpu.VMEM((EMBED_DIM, SLAB), jnp.float32),
            pltpu.VMEM((EMBED_DIM, SLAB), jnp.float32),
            pltpu.VMEM((SLAB, SLAB), jnp.float32),
            pltpu.VMEM((SLAB, SLAB), jnp.float32),
            pltpu.VMEM((TAIL, EMBED_DIM), jnp.float32),
            pltpu.SemaphoreType.DMA,
            pltpu.SemaphoreType.DMA,
            pltpu.SemaphoreType.DMA,
            pltpu.SemaphoreType.DMA,
            pltpu.SemaphoreType.DMA,
        ],
        compiler_params=pltpu.CompilerParams(needs_layout_passes=False),
    )(tt, tail)

    y5 = pl.kernel(
        _gather_body,
        out_type=jax.ShapeDtypeStruct(
            (SEQ_LEN, EMBED_DIM // 8, NW, 8, SLAB), jnp.float32),
        mesh=mesh,
        scratch_types=[
            pltpu.VMEM((SEQ_LEN, SLAB), jnp.int32),
            pltpu.VMEM((SEQ_LEN, EMBED_DIM), jnp.float32),
            pltpu.VMEM((SLAB, SLAB), jnp.float32),
            pltpu.VMEM((SLAB, SLAB), jnp.float32),
            pltpu.VMEM((SLAB, SLAB), jnp.float32),
            pltpu.VMEM((EMBED_DIM, SLAB), jnp.float32),
            pltpu.VMEM((EMBED_DIM, SLAB), jnp.float32),
            pltpu.SemaphoreType.DMA,
            pltpu.SemaphoreType.DMA,
            pltpu.SemaphoreType.DMA,
            pltpu.SemaphoreType.DMA,
            pltpu.SemaphoreType.DMA,
        ],
        compiler_params=pltpu.CompilerParams(needs_layout_passes=False),
    )(xt, t2, pos)
    return y5


def kernel(x_in, table):
    xt = x_in.astype(jnp.int32).T
    tt = table.T
    tail = table[NFULL * SLAB:]
    pos = jnp.asarray(_positional_encoding(SEQ_LEN, EMBED_DIM))
    y5 = _run(xt, tt, tail, pos)
    return y5.transpose(2, 4, 0, 1, 3).reshape(BATCH, SEQ_LEN, EMBED_DIM)


# speedup vs baseline: 2.1729x; 1.0831x over previous
import jax
import jax.numpy as jnp
import numpy as np
from jax import lax
from jax.experimental import pallas as pl
from jax.experimental.pallas import tpu as pltpu
from jax.experimental.pallas import tpu_sc as plsc

NOTES_POOL_SIZE = 1000000
EMBED_DIM = 64
BATCH = 4096
SEQ_LEN = 200

NC = 2
NS = 16
NW = NC * NS
SLAB = 128
NFULL = NOTES_POOL_SIZE // SLAB
TAIL = NOTES_POOL_SIZE - NFULL * SLAB
T2_ROWS = (NFULL + 1) * SLAB
TAIL_WORKER = NFULL % NW


def _positional_encoding(max_pos, embed_dim):
    pos = np.arange(max_pos)[:, np.newaxis]
    i = np.arange(embed_dim)[np.newaxis, :]
    angle_rates = 1.0 / np.power(10000, 2 * (i // 2) / np.float32(embed_dim))
    angle_rads = pos * angle_rates
    angle_rads[:, 0::2] = np.sin(angle_rads[:, 0::2])
    angle_rads[:, 1::2] = np.cos(angle_rads[:, 1::2])
    return angle_rads.astype(np.float32)


def _splat(v):
    return jnp.full((16,), v, jnp.int32)


def _transpose_body(tt_hbm, tail_hbm, t2_hbm, slab0, slab1, tbuf0, tbuf1,
                    tail_v, isem0, isem1, osem0, osem1, tsem):
    wid = lax.axis_index("s") * NC + lax.axis_index("c")
    slab = [slab0, slab1]
    tbuf = [tbuf0, tbuf1]
    isem = [isem0, isem1]
    osem = [osem0, osem1]
    iota = lax.iota(jnp.int32, 16)
    iotas = [iota + 16 * k for k in range(EMBED_DIM // 16)]

    nmine = (NFULL - wid + NW - 1) // NW

    def col_ds(j):
        return pl.ds(pl.multiple_of(j * SLAB, SLAB), SLAB)

    def start_in(i, p):
        j = wid + NW * i
        pltpu.async_copy(tt_hbm.at[:, col_ds(j)], slab[p], isem[p])

    def wait_in(p):
        pltpu.make_async_copy(tt_hbm.at[:, col_ds(0)], slab[p],
                              isem[p]).wait()

    def start_out(i, p):
        j = wid + NW * i
        pltpu.async_copy(tbuf[p], t2_hbm.at[col_ds(j)], osem[p])

    def wait_out(p):
        pltpu.make_async_copy(tbuf[p], t2_hbm.at[col_ds(0)], osem[p]).wait()

    diags = [lax.rem(iota + i, 16) for i in range(16)]

    def transpose_slab(p):
        def cblock(m, carry):
            cvec = iota + 16 * m
            for k in range(EMBED_DIM // 16):
                for i in range(16):
                    dvec = diags[i] + (16 * k)
                    v = plsc.load_gather(slab[p], [dvec, cvec])
                    plsc.store_scatter(tbuf[p], [cvec, dvec], v)
            return carry
        lax.fori_loop(0, SLAB // 16, cblock, 0)

    @pl.when(nmine > 0)
    def _():
        start_in(0, 0)

    def step(i, carry):
        for p in range(2):
            @pl.when(lax.rem(i, 2) == p)
            def _(p=p):
                wait_in(p)

                @pl.when(i + 1 < nmine)
                def _():
                    start_in(i + 1, 1 - p)

                @pl.when(i >= 2)
                def _():
                    wait_out(p)

                transpose_slab(p)
                start_out(i, p)
        return carry

    lax.fori_loop(0, nmine, step, 0)

    wait_out(0)
    wait_out(1)

    @pl.when(wid == TAIL_WORKER)
    def _():
        pltpu.async_copy(tail_hbm, tail_v, tsem)
        pltpu.make_async_copy(tail_hbm, tail_v, tsem).wait()

        def row(r, carry):
            for k in range(EMBED_DIM // 16):
                s16 = pl.ds(16 * k, 16)
                tbuf0[r, s16] = tail_v[r, s16]
            return carry
        lax.fori_loop(0, TAIL, row, 0)
        pltpu.async_copy(tbuf0.at[pl.ds(0, TAIL)],
                         t2_hbm.at[pl.ds(NFULL * SLAB, TAIL)], tsem)
        pltpu.make_async_copy(tbuf0.at[pl.ds(0, TAIL)],
                              t2_hbm.at[pl.ds(NFULL * SLAB, TAIL)],
                              tsem).wait()


def _gather_body(xt_hbm, t2_hbm, pos_hbm, out_hbm,
                 idx_v, pos_v, g0, g1, ob0, ob1,
                 gsem0, gsem1, osem0, osem1):
    wid = lax.axis_index("s") * NC + lax.axis_index("c")
    g = [g0, g1]
    ob = [ob0, ob1]
    gsem = [gsem0, gsem1]
    osem = [osem0, osem1]
    iota = lax.iota(jnp.int32, 16)
    diags = [lax.rem(iota + i, 16) for i in range(16)]

    bds = pl.ds(pl.multiple_of(wid * SLAB, SLAB), SLAB)
    pltpu.sync_copy(xt_hbm.at[:, bds], idx_v)
    pltpu.sync_copy(pos_hbm, pos_v)

    def fire_gather(s, q):
        pltpu.async_copy(t2_hbm.at[idx_v.at[s]], g[q], gsem[q])

    def wait_gather(s, q):
        pltpu.make_async_copy(t2_hbm.at[idx_v.at[s]], g[q], gsem[q]).wait()

    def wait_store(s, p):
        for dt in range(EMBED_DIM // 8):
            pltpu.make_async_copy(ob[p].at[pl.ds(8 * dt, 8)],
                                  out_hbm.at[s, dt, wid], osem[p]).wait()

    fire_gather(0, 0)

    def step(s, carry):
        p0 = lax.rem(s, 2)
        for p in range(2):
            @pl.when(p0 == p)
            def _(p=p):
                wait_gather(s, p)

                @pl.when(s + 1 < SEQ_LEN)
                def _():
                    fire_gather(s + 1, 1 - p)

                @pl.when(s >= 2)
                def _():
                    wait_store(s - 2, p)

                pregs = [pos_v[s, pl.ds(16 * k, 16)]
                         for k in range(EMBED_DIM // 16)]

                def brow(b4, carry):
                    for bb in range(4):
                        bl = 4 * b4 + bb
                        for k in range(EMBED_DIM // 16):
                            plsc.addupdate(
                                g[p].at[bl, pl.ds(16 * k, 16)], pregs[k])
                    return carry
                lax.fori_loop(0, SLAB // 4, brow, 0)

                def bblock(m, carry):
                    blvec = iota + 16 * m
                    for k in range(EMBED_DIM // 16):
                        for i in range(16):
                            dvec = diags[i] + (16 * k)
                            v = plsc.load_gather(g[p], [blvec, dvec])
                            plsc.store_scatter(ob[p], [dvec, blvec], v)
                    return carry
                lax.fori_loop(0, SLAB // 16, bblock, 0)

                for dt in range(EMBED_DIM // 8):
                    pltpu.async_copy(ob[p].at[pl.ds(8 * dt, 8)],
                                     out_hbm.at[s, dt, wid], osem[p])
        return carry

    lax.fori_loop(0, SEQ_LEN, step, 0)
    wait_store(SEQ_LEN - 2, 0)
    wait_store(SEQ_LEN - 1, 1)


@jax.jit
def _run(xt, tt, tail, pos):
    mesh = plsc.VectorSubcoreMesh(core_axis_name="c", subcore_axis_name="s")
    t2 = pl.kernel(
        _transpose_body,
        out_type=jax.ShapeDtypeStruct((T2_ROWS, SLAB), jnp.float32),
        mesh=mesh,
        scratch_types=[
            pltpu.VMEM((EMBED_DIM, SLAB), jnp.float32),
            pltpu.VMEM((EMBED_DIM, SLAB), jnp.float32),
            pltpu.VMEM((SLAB, SLAB), jnp.float32),
            pltpu.VMEM((SLAB, SLAB), jnp.float32),
            pltpu.VMEM((TAIL, EMBED_DIM), jnp.float32),
            pltpu.SemaphoreType.DMA,
            pltpu.SemaphoreType.DMA,
            pltpu.SemaphoreType.DMA,
            pltpu.SemaphoreType.DMA,
            pltpu.SemaphoreType.DMA,
        ],
        compiler_params=pltpu.CompilerParams(needs_layout_passes=False),
    )(tt, tail)

    y5 = pl.kernel(
        _gather_body,
        out_type=jax.ShapeDtypeStruct(
            (SEQ_LEN, EMBED_DIM // 8, NW, 8, SLAB), jnp.float32),
        mesh=mesh,
        scratch_types=[
            pltpu.VMEM((SEQ_LEN, SLAB), jnp.int32),
            pltpu.VMEM((SEQ_LEN, EMBED_DIM), jnp.float32),
            pltpu.VMEM((SLAB, SLAB), jnp.float32),
            pltpu.VMEM((SLAB, SLAB), jnp.float32),
            pltpu.VMEM((EMBED_DIM, SLAB), jnp.float32),
            pltpu.VMEM((EMBED_DIM, SLAB), jnp.float32),
            pltpu.SemaphoreType.DMA,
            pltpu.SemaphoreType.DMA,
            pltpu.SemaphoreType.DMA,
            pltpu.SemaphoreType.DMA,
        ],
        compiler_params=pltpu.CompilerParams(needs_layout_passes=False),
    )(xt, t2, pos)
    return y5


def kernel(x_in, table):
    xt = x_in.astype(jnp.int32).T
    tt = table.T
    tail = table[NFULL * SLAB:]
    pos = jnp.asarray(_positional_encoding(SEQ_LEN, EMBED_DIM))
    y5 = _run(xt, tt, tail, pos)
    return y5.transpose(2, 4, 0, 1, 3).reshape(BATCH, SEQ_LEN, EMBED_DIM)


# speedup vs baseline: 2.1871x; 1.0066x over previous
import jax
import jax.numpy as jnp
import numpy as np
from jax import lax
from jax.experimental import pallas as pl
from jax.experimental.pallas import tpu as pltpu
from jax.experimental.pallas import tpu_sc as plsc

NOTES_POOL_SIZE = 1000000
EMBED_DIM = 64
BATCH = 4096
SEQ_LEN = 200

NC = 2
NS = 16
NW = NC * NS
SLAB = 128
NFULL = NOTES_POOL_SIZE // SLAB
TAIL = NOTES_POOL_SIZE - NFULL * SLAB
T2_ROWS = (NFULL + 1) * SLAB
TAIL_WORKER = NFULL % NW


def _positional_encoding(max_pos, embed_dim):
    pos = np.arange(max_pos)[:, np.newaxis]
    i = np.arange(embed_dim)[np.newaxis, :]
    angle_rates = 1.0 / np.power(10000, 2 * (i // 2) / np.float32(embed_dim))
    angle_rads = pos * angle_rates
    angle_rads[:, 0::2] = np.sin(angle_rads[:, 0::2])
    angle_rads[:, 1::2] = np.cos(angle_rads[:, 1::2])
    return angle_rads.astype(np.float32)


def _splat(v):
    return jnp.full((16,), v, jnp.int32)


def _transpose_body(tt_hbm, tail_hbm, t2_hbm, slab0, slab1, tbuf0, tbuf1,
                    tail_v, isem0, isem1, osem0, osem1, tsem):
    wid = lax.axis_index("s") * NC + lax.axis_index("c")
    slab = [slab0, slab1]
    tbuf = [tbuf0, tbuf1]
    isem = [isem0, isem1]
    osem = [osem0, osem1]
    iota = lax.iota(jnp.int32, 16)
    iotas = [iota + 16 * k for k in range(EMBED_DIM // 16)]

    nmine = (NFULL - wid + NW - 1) // NW

    def col_ds(j):
        return pl.ds(pl.multiple_of(j * SLAB, SLAB), SLAB)

    def start_in(i, p):
        j = wid + NW * i
        pltpu.async_copy(tt_hbm.at[:, col_ds(j)], slab[p], isem[p])

    def wait_in(p):
        pltpu.make_async_copy(tt_hbm.at[:, col_ds(0)], slab[p],
                              isem[p]).wait()

    def start_out(i, p):
        j = wid + NW * i
        pltpu.async_copy(tbuf[p], t2_hbm.at[col_ds(j)], osem[p])

    def wait_out(p):
        pltpu.make_async_copy(tbuf[p], t2_hbm.at[col_ds(0)], osem[p]).wait()

    diags = [lax.rem(iota + i, 16) for i in range(16)]

    def transpose_slab(p):
        def cblock(m, carry):
            cvec = iota + 16 * m
            for k in range(EMBED_DIM // 16):
                for i in range(16):
                    dvec = diags[i] + (16 * k)
                    v = plsc.load_gather(slab[p], [dvec, cvec])
                    plsc.store_scatter(tbuf[p], [cvec, dvec], v)
            return carry
        lax.fori_loop(0, SLAB // 16, cblock, 0)

    @pl.when(nmine > 0)
    def _():
        start_in(0, 0)

    def step(i, carry):
        for p in range(2):
            @pl.when(lax.rem(i, 2) == p)
            def _(p=p):
                wait_in(p)

                @pl.when(i + 1 < nmine)
                def _():
                    start_in(i + 1, 1 - p)

                @pl.when(i >= 2)
                def _():
                    wait_out(p)

                transpose_slab(p)
                start_out(i, p)
        return carry

    lax.fori_loop(0, nmine, step, 0)

    wait_out(0)
    wait_out(1)

    @pl.when(wid == TAIL_WORKER)
    def _():
        pltpu.async_copy(tail_hbm, tail_v, tsem)
        pltpu.make_async_copy(tail_hbm, tail_v, tsem).wait()

        def row(r, carry):
            for k in range(EMBED_DIM // 16):
                s16 = pl.ds(16 * k, 16)
                tbuf0[r, s16] = tail_v[r, s16]
            return carry
        lax.fori_loop(0, TAIL, row, 0)
        pltpu.async_copy(tbuf0.at[pl.ds(0, TAIL)],
                         t2_hbm.at[pl.ds(NFULL * SLAB, TAIL)], tsem)
        pltpu.make_async_copy(tbuf0.at[pl.ds(0, TAIL)],
                              t2_hbm.at[pl.ds(NFULL * SLAB, TAIL)],
                              tsem).wait()


def _gather_body(xt_hbm, t2_hbm, pos_hbm, out_hbm,
                 idx_v, pos_v, g0, g1, ob0, ob1,
                 gsem0, gsem1, osem0, osem1):
    wid = lax.axis_index("s") * NC + lax.axis_index("c")
    g = [g0, g1]
    ob = [ob0, ob1]
    gsem = [gsem0, gsem1]
    osem = [osem0, osem1]
    iota = lax.iota(jnp.int32, 16)
    diags = [lax.rem(iota + i, 16) for i in range(16)]

    bds = pl.ds(pl.multiple_of(wid * SLAB, SLAB), SLAB)
    pltpu.sync_copy(xt_hbm.at[:, bds], idx_v)
    pltpu.sync_copy(pos_hbm, pos_v)

    def fire_gather(s, q):
        pltpu.async_copy(t2_hbm.at[idx_v.at[s]], g[q], gsem[q])

    def wait_gather(s, q):
        pltpu.make_async_copy(t2_hbm.at[idx_v.at[s]], g[q], gsem[q]).wait()

    def wait_store(s, p):
        pltpu.make_async_copy(ob[p], out_hbm.at[s, :, wid], osem[p]).wait()

    fire_gather(0, 0)

    def step(s, carry):
        p0 = lax.rem(s, 2)
        for p in range(2):
            @pl.when(p0 == p)
            def _(p=p):
                wait_gather(s, p)

                @pl.when(s + 1 < SEQ_LEN)
                def _():
                    fire_gather(s + 1, 1 - p)

                @pl.when(s >= 2)
                def _():
                    wait_store(s - 2, p)

                pregs = [pos_v[s, pl.ds(16 * k, 16)]
                         for k in range(EMBED_DIM // 16)]

                def brow(b4, carry):
                    for bb in range(4):
                        bl = 4 * b4 + bb
                        for k in range(EMBED_DIM // 16):
                            plsc.addupdate(
                                g[p].at[bl, pl.ds(16 * k, 16)], pregs[k])
                    return carry
                lax.fori_loop(0, SLAB // 4, brow, 0)

                def bblock(m, carry):
                    blvec = iota + 16 * m
                    for k in range(EMBED_DIM // 16):
                        for i in range(16):
                            dvec = diags[i] + (16 * k)
                            v = plsc.load_gather(g[p], [blvec, dvec])
                            plsc.store_scatter(
                                ob[p],
                                [dvec >> 3, dvec & 7, blvec], v)
                    return carry
                lax.fori_loop(0, SLAB // 16, bblock, 0)

                pltpu.async_copy(ob[p], out_hbm.at[s, :, wid], osem[p])
        return carry

    lax.fori_loop(0, SEQ_LEN, step, 0)
    wait_store(SEQ_LEN - 2, 0)
    wait_store(SEQ_LEN - 1, 1)


@jax.jit
def _run(xt, tt, tail, pos):
    mesh = plsc.VectorSubcoreMesh(core_axis_name="c", subcore_axis_name="s")
    t2 = pl.kernel(
        _transpose_body,
        out_type=jax.ShapeDtypeStruct((T2_ROWS, SLAB), jnp.float32),
        mesh=mesh,
        scratch_types=[
            pltpu.VMEM((EMBED_DIM, SLAB), jnp.float32),
            pltpu.VMEM((EMBED_DIM, SLAB), jnp.float32),
            pltpu.VMEM((SLAB, SLAB), jnp.float32),
            pltpu.VMEM((SLAB, SLAB), jnp.float32),
            pltpu.VMEM((TAIL, EMBED_DIM), jnp.float32),
            pltpu.SemaphoreType.DMA,
            pltpu.SemaphoreType.DMA,
            pltpu.SemaphoreType.DMA,
            pltpu.SemaphoreType.DMA,
            pltpu.SemaphoreType.DMA,
        ],
        compiler_params=pltpu.CompilerParams(needs_layout_passes=False),
    )(tt, tail)

    y5 = pl.kernel(
        _gather_body,
        out_type=jax.ShapeDtypeStruct(
            (SEQ_LEN, EMBED_DIM // 8, NW, 8, SLAB), jnp.float32),
        mesh=mesh,
        scratch_types=[
            pltpu.VMEM((SEQ_LEN, SLAB), jnp.int32),
            pltpu.VMEM((SEQ_LEN, EMBED_DIM), jnp.float32),
            pltpu.VMEM((SLAB, SLAB), jnp.float32),
            pltpu.VMEM((SLAB, SLAB), jnp.float32),
            pltpu.VMEM((EMBED_DIM // 8, 8, SLAB), jnp.float32),
            pltpu.VMEM((EMBED_DIM // 8, 8, SLAB), jnp.float32),
            pltpu.SemaphoreType.DMA,
            pltpu.SemaphoreType.DMA,
            pltpu.SemaphoreType.DMA,
            pltpu.SemaphoreType.DMA,
        ],
        compiler_params=pltpu.CompilerParams(needs_layout_passes=False),
    )(xt, t2, pos)
    return y5


def kernel(x_in, table):
    xt = x_in.astype(jnp.int32).T
    tt = table.T
    tail = table[NFULL * SLAB:]
    pos = jnp.asarray(_positional_encoding(SEQ_LEN, EMBED_DIM))
    y5 = _run(xt, tt, tail, pos)
    return y5.transpose(2, 4, 0, 1, 3).reshape(BATCH, SEQ_LEN, EMBED_DIM)


# speedup vs baseline: 2.1938x; 1.0030x over previous
import jax
import jax.numpy as jnp
import numpy as np
from jax import lax
from jax.experimental import pallas as pl
from jax.experimental.pallas import tpu as pltpu
from jax.experimental.pallas import tpu_sc as plsc

NOTES_POOL_SIZE = 1000000
EMBED_DIM = 64
BATCH = 4096
SEQ_LEN = 200

NC = 2
NS = 16
NW = NC * NS
SLAB = 128
NFULL = NOTES_POOL_SIZE // SLAB
TAIL = NOTES_POOL_SIZE - NFULL * SLAB
T2_ROWS = (NFULL + 1) * SLAB
TAIL_WORKER = NFULL % NW


def _positional_encoding(max_pos, embed_dim):
    pos = np.arange(max_pos)[:, np.newaxis]
    i = np.arange(embed_dim)[np.newaxis, :]
    angle_rates = 1.0 / np.power(10000, 2 * (i // 2) / np.float32(embed_dim))
    angle_rads = pos * angle_rates
    angle_rads[:, 0::2] = np.sin(angle_rads[:, 0::2])
    angle_rads[:, 1::2] = np.cos(angle_rads[:, 1::2])
    return angle_rads.astype(np.float32)


def _splat(v):
    return jnp.full((16,), v, jnp.int32)


def _transpose_body(tt_hbm, tail_hbm, t2_hbm, slab0, slab1, slab2,
                    tbuf0, tbuf1, tbuf2, tail_v,
                    isem0, isem1, isem2, osem0, osem1, osem2, tsem):
    wid = lax.axis_index("s") * NC + lax.axis_index("c")
    slab = [slab0, slab1, slab2]
    tbuf = [tbuf0, tbuf1, tbuf2]
    isem = [isem0, isem1, isem2]
    osem = [osem0, osem1, osem2]
    iota = lax.iota(jnp.int32, 16)
    iotas = [iota + 16 * k for k in range(EMBED_DIM // 16)]

    nmine = (NFULL - wid + NW - 1) // NW

    def col_ds(j):
        return pl.ds(pl.multiple_of(j * SLAB, SLAB), SLAB)

    def start_in(i, p):
        j = wid + NW * i
        pltpu.async_copy(tt_hbm.at[:, col_ds(j)], slab[p], isem[p])

    def wait_in(p):
        pltpu.make_async_copy(tt_hbm.at[:, col_ds(0)], slab[p],
                              isem[p]).wait()

    def start_out(i, p):
        j = wid + NW * i
        pltpu.async_copy(tbuf[p], t2_hbm.at[col_ds(j)], osem[p])

    def wait_out(p):
        pltpu.make_async_copy(tbuf[p], t2_hbm.at[col_ds(0)], osem[p]).wait()

    diags = [lax.rem(iota + i, 16) for i in range(16)]

    def transpose_slab(p):
        def cblock(m, carry):
            cvec = iota + 16 * m
            for k in range(EMBED_DIM // 16):
                for i in range(16):
                    dvec = diags[i] + (16 * k)
                    v = plsc.load_gather(slab[p], [dvec, cvec])
                    plsc.store_scatter(tbuf[p], [cvec, dvec], v)
            return carry
        lax.fori_loop(0, SLAB // 16, cblock, 0)

    start_in(0, 0)
    start_in(1, 1)

    def step(i, carry):
        for p in range(3):
            @pl.when(lax.rem(i, 3) == p)
            def _(p=p):
                wait_in(p)

                @pl.when(i + 2 < nmine)
                def _():
                    start_in(i + 2, (p + 2) % 3)

                @pl.when(i >= 3)
                def _():
                    wait_out(p)

                transpose_slab(p)
                start_out(i, p)
        return carry

    lax.fori_loop(0, nmine, step, 0)

    wait_out(0)
    wait_out(1)
    wait_out(2)

    @pl.when(wid == TAIL_WORKER)
    def _():
        pltpu.async_copy(tail_hbm, tail_v, tsem)
        pltpu.make_async_copy(tail_hbm, tail_v, tsem).wait()

        def row(r, carry):
            for k in range(EMBED_DIM // 16):
                s16 = pl.ds(16 * k, 16)
                tbuf0[r, s16] = tail_v[r, s16]
            return carry
        lax.fori_loop(0, TAIL, row, 0)
        pltpu.async_copy(tbuf0.at[pl.ds(0, TAIL)],
                         t2_hbm.at[pl.ds(NFULL * SLAB, TAIL)], tsem)
        pltpu.make_async_copy(tbuf0.at[pl.ds(0, TAIL)],
                              t2_hbm.at[pl.ds(NFULL * SLAB, TAIL)],
                              tsem).wait()


def _gather_body(xt_hbm, t2_hbm, pos_hbm, out_hbm,
                 idx_v, pos_v, g0, g1, g2, ob0, ob1, ob2,
                 gsem0, gsem1, gsem2, osem0, osem1, osem2):
    wid = lax.axis_index("s") * NC + lax.axis_index("c")
    g = [g0, g1, g2]
    ob = [ob0, ob1, ob2]
    gsem = [gsem0, gsem1, gsem2]
    osem = [osem0, osem1, osem2]
    iota = lax.iota(jnp.int32, 16)
    diags = [lax.rem(iota + i, 16) for i in range(16)]

    bds = pl.ds(pl.multiple_of(wid * SLAB, SLAB), SLAB)
    pltpu.sync_copy(xt_hbm.at[:, bds], idx_v)
    pltpu.sync_copy(pos_hbm, pos_v)

    def fire_gather(s, q):
        pltpu.async_copy(t2_hbm.at[idx_v.at[s]], g[q], gsem[q])

    def wait_gather(s, q):
        pltpu.make_async_copy(t2_hbm.at[idx_v.at[s]], g[q], gsem[q]).wait()

    def wait_store(s, p):
        pltpu.make_async_copy(ob[p], out_hbm.at[s, :, wid], osem[p]).wait()

    fire_gather(0, 0)
    fire_gather(1, 1)

    def step(s, carry):
        p0 = lax.rem(s, 3)
        for p in range(3):
            @pl.when(p0 == p)
            def _(p=p):
                wait_gather(s, p)

                @pl.when(s + 2 < SEQ_LEN)
                def _():
                    fire_gather(s + 2, (p + 2) % 3)

                @pl.when(s >= 3)
                def _():
                    wait_store(s - 3, p)

                pregs = [pos_v[s, pl.ds(16 * k, 16)]
                         for k in range(EMBED_DIM // 16)]

                def brow(b4, carry):
                    for bb in range(4):
                        bl = 4 * b4 + bb
                        for k in range(EMBED_DIM // 16):
                            plsc.addupdate(
                                g[p].at[bl, pl.ds(16 * k, 16)], pregs[k])
                    return carry
                lax.fori_loop(0, SLAB // 4, brow, 0)

                def bblock(m, carry):
                    blvec = iota + 16 * m
                    for k in range(EMBED_DIM // 16):
                        for i in range(16):
                            dvec = diags[i] + (16 * k)
                            v = plsc.load_gather(g[p], [blvec, dvec])
                            plsc.store_scatter(
                                ob[p],
                                [dvec >> 3, dvec & 7, blvec], v)
                    return carry
                lax.fori_loop(0, SLAB // 16, bblock, 0)

                pltpu.async_copy(ob[p], out_hbm.at[s, :, wid], osem[p])
        return carry

    lax.fori_loop(0, SEQ_LEN, step, 0)
    wait_store(SEQ_LEN - 3, (SEQ_LEN - 3) % 3)
    wait_store(SEQ_LEN - 2, (SEQ_LEN - 2) % 3)
    wait_store(SEQ_LEN - 1, (SEQ_LEN - 1) % 3)


@jax.jit
def _run(xt, tt, tail, pos):
    mesh = plsc.VectorSubcoreMesh(core_axis_name="c", subcore_axis_name="s")
    t2 = pl.kernel(
        _transpose_body,
        out_type=jax.ShapeDtypeStruct((T2_ROWS, SLAB), jnp.float32),
        mesh=mesh,
        scratch_types=[
            pltpu.VMEM((EMBED_DIM, SLAB), jnp.float32),
            pltpu.VMEM((EMBED_DIM, SLAB), jnp.float32),
            pltpu.VMEM((EMBED_DIM, SLAB), jnp.float32),
            pltpu.VMEM((SLAB, SLAB), jnp.float32),
            pltpu.VMEM((SLAB, SLAB), jnp.float32),
            pltpu.VMEM((SLAB, SLAB), jnp.float32),
            pltpu.VMEM((TAIL, EMBED_DIM), jnp.float32),
            pltpu.SemaphoreType.DMA,
            pltpu.SemaphoreType.DMA,
            pltpu.SemaphoreType.DMA,
            pltpu.SemaphoreType.DMA,
            pltpu.SemaphoreType.DMA,
            pltpu.SemaphoreType.DMA,
            pltpu.SemaphoreType.DMA,
        ],
        compiler_params=pltpu.CompilerParams(needs_layout_passes=False),
    )(tt, tail)

    y5 = pl.kernel(
        _gather_body,
        out_type=jax.ShapeDtypeStruct(
            (SEQ_LEN, EMBED_DIM // 8, NW, 8, SLAB), jnp.float32),
        mesh=mesh,
        scratch_types=[
            pltpu.VMEM((SEQ_LEN, SLAB), jnp.int32),
            pltpu.VMEM((SEQ_LEN, EMBED_DIM), jnp.float32),
            pltpu.VMEM((SLAB, SLAB), jnp.float32),
            pltpu.VMEM((SLAB, SLAB), jnp.float32),
            pltpu.VMEM((SLAB, SLAB), jnp.float32),
            pltpu.VMEM((EMBED_DIM // 8, 8, SLAB), jnp.float32),
            pltpu.VMEM((EMBED_DIM // 8, 8, SLAB), jnp.float32),
            pltpu.VMEM((EMBED_DIM // 8, 8, SLAB), jnp.float32),
            pltpu.SemaphoreType.DMA,
            pltpu.SemaphoreType.DMA,
            pltpu.SemaphoreType.DMA,
            pltpu.SemaphoreType.DMA,
            pltpu.SemaphoreType.DMA,
            pltpu.SemaphoreType.DMA,
        ],
        compiler_params=pltpu.CompilerParams(needs_layout_passes=False),
    )(xt, t2, pos)
    return y5


def kernel(x_in, table):
    xt = x_in.astype(jnp.int32).T
    tt = table.T
    tail = table[NFULL * SLAB:]
    pos = jnp.asarray(_positional_encoding(SEQ_LEN, EMBED_DIM))
    y5 = _run(xt, tt, tail, pos)
    return y5.transpose(2, 4, 0, 1, 3).reshape(BATCH, SEQ_LEN, EMBED_DIM)


# speedup vs baseline: 2.2239x; 1.0138x over previous
import jax
import jax.numpy as jnp
import numpy as np
from jax import lax
from jax.experimental import pallas as pl
from jax.experimental.pallas import tpu as pltpu
from jax.experimental.pallas import tpu_sc as plsc

NOTES_POOL_SIZE = 1000000
EMBED_DIM = 64
BATCH = 4096
SEQ_LEN = 200

NC = 2
NS = 16
NW = NC * NS
ROWS_PER_W = BATCH // NW
NCHUNK = 5
CHUNK = SEQ_LEN // NCHUNK
NBUF = 3


def _positional_encoding(max_pos, embed_dim):
    pos = np.arange(max_pos)[:, np.newaxis]
    i = np.arange(embed_dim)[np.newaxis, :]
    angle_rates = 1.0 / np.power(10000, 2 * (i // 2) / np.float32(embed_dim))
    angle_rads = pos * angle_rates
    angle_rads[:, 0::2] = np.sin(angle_rads[:, 0::2])
    angle_rads[:, 1::2] = np.cos(angle_rads[:, 1::2])
    return angle_rads.astype(np.float32)


def _body(x_hbm, table_hbm, pos_hbm, out_hbm,
          pos_v, idx_v, rows0, rows1, rows2,
          psem, xsem, gsem0, gsem1, gsem2, ssem0, ssem1, ssem2):
    rows = [rows0, rows1, rows2]
    gsem = [gsem0, gsem1, gsem2]
    ssem = [ssem0, ssem1, ssem2]

    wid = lax.axis_index("s") * NC + lax.axis_index("c")
    base = wid * ROWS_PER_W

    pltpu.async_copy(pos_hbm, pos_v, psem)
    pltpu.async_copy(x_hbm.at[pl.ds(base, ROWS_PER_W)], idx_v, xsem)
    pltpu.make_async_copy(pos_hbm, pos_v, psem).wait()
    pltpu.make_async_copy(x_hbm.at[pl.ds(base, ROWS_PER_W)], idx_v,
                          xsem).wait()

    def fire_gathers(b, q):
        for h in range(NCHUNK):
            pltpu.async_copy(table_hbm.at[idx_v.at[b, h]],
                             rows[q].at[pl.ds(h * CHUNK, CHUNK)], gsem[q])

    def wait_gather(b, q):
        for h in range(NCHUNK):
            pltpu.make_async_copy(table_hbm.at[idx_v.at[b, h]],
                                  rows[q].at[pl.ds(h * CHUNK, CHUNK)],
                                  gsem[q]).wait()

    def wait_store(q):
        pltpu.make_async_copy(rows[q], out_hbm.at[base], ssem[q]).wait()

    def add_pos(q):
        def inner(i, carry):
            for ii in range(8):
                r = 8 * i + ii
                for j in range(EMBED_DIM // 16):
                    s = pl.ds(16 * j, 16)
                    plsc.addupdate(rows[q].at[r, s], pos_v[r, s])
            return carry
        lax.fori_loop(0, SEQ_LEN // 8, inner, 0)

    fire_gathers(0, 0)

    def step(b, carry):
        q0 = lax.rem(b, NBUF)
        for q in range(NBUF):
            @pl.when(q0 == q)
            def _(q=q):
                wait_gather(b, q)

                @pl.when(b >= 2)
                def _():
                    wait_store((q + 1) % NBUF)

                @pl.when(b < ROWS_PER_W - 1)
                def _():
                    fire_gathers(b + 1, (q + 1) % NBUF)

                add_pos(q)
                pltpu.async_copy(rows[q], out_hbm.at[base + b], ssem[q])
        return carry

    lax.fori_loop(0, ROWS_PER_W, step, 0)

    wait_store((ROWS_PER_W - 2) % NBUF)
    wait_store((ROWS_PER_W - 1) % NBUF)


@jax.jit
def _run(x3, table, pos):
    mesh = plsc.VectorSubcoreMesh(core_axis_name="c", subcore_axis_name="s")
    k = pl.kernel(
        _body,
        out_type=jax.ShapeDtypeStruct((BATCH, SEQ_LEN, EMBED_DIM),
                                      jnp.float32),
        mesh=mesh,
        scratch_types=[
            pltpu.VMEM((SEQ_LEN, EMBED_DIM), jnp.float32),
            pltpu.VMEM((ROWS_PER_W, NCHUNK, CHUNK), jnp.int32),
            pltpu.VMEM((SEQ_LEN, EMBED_DIM), jnp.float32),
            pltpu.VMEM((SEQ_LEN, EMBED_DIM), jnp.float32),
            pltpu.VMEM((SEQ_LEN, EMBED_DIM), jnp.float32),
            pltpu.SemaphoreType.DMA,
            pltpu.SemaphoreType.DMA,
            pltpu.SemaphoreType.DMA,
            pltpu.SemaphoreType.DMA,
            pltpu.SemaphoreType.DMA,
            pltpu.SemaphoreType.DMA,
            pltpu.SemaphoreType.DMA,
            pltpu.SemaphoreType.DMA,
        ],
        compiler_params=pltpu.CompilerParams(use_tc_tiling_on_sc=False),
    )
    return k(x3, table, pos)


def kernel(x_in, table):
    x3 = x_in.astype(jnp.int32).reshape(BATCH, NCHUNK, CHUNK)
    pos = jnp.asarray(_positional_encoding(SEQ_LEN, EMBED_DIM))
    return _run(x3, table, pos)
